# baseline (speedup 1.0000x reference)
import jax
import jax.numpy as jnp
from jax import lax
from jax.experimental import pallas as pl
from jax.experimental.pallas import tpu as pltpu
from jax.experimental.pallas import tpu_sc as plsc

H = 768
NW = 32
SEG = 1024
CAP = 128
L = 16

_SC_MESH = plsc.VectorSubcoreMesh(
    core_axis_name="c", subcore_axis_name="s", num_cores=2, num_subcores=16
)


def _sc_compact_gather(mask_hbm, x_hbm, idx_hbm, cnt_hbm, xc_hbm,
                       mask_v, idxl_v, idxg_v, rows_v, cnt_v, sem):
    wid = lax.axis_index("s") * 2 + lax.axis_index("c")
    base = wid * SEG
    pltpu.sync_copy(mask_hbm.at[pl.ds(base, SEG)], mask_v)

    zeros = jnp.zeros((L,), jnp.int32)
    for i in range(CAP // L):
        idxl_v[pl.ds(i * L, L)] = zeros

    def body(c, off):
        mchunk = mask_v[pl.ds(c * L, L)]
        sel = mchunk == 1
        seli = sel.astype(jnp.int32)
        pos = off + plsc.cumsum(seli) - 1
        okay = sel & (pos < CAP)
        posc = jnp.minimum(pos, CAP - 1)
        localpos = c * L + lax.iota(jnp.int32, L)
        plsc.store_scatter(idxl_v, [posc], localpos, mask=okay)
        return off + jnp.sum(seli)

    n = lax.fori_loop(0, SEG // L, body, jnp.int32(0))

    for i in range(CAP // L):
        idxg_v[pl.ds(i * L, L)] = idxl_v[pl.ds(i * L, L)] + base

    pltpu.async_copy(x_hbm.at[idxg_v], rows_v, sem).wait()
    pltpu.sync_copy(rows_v, xc_hbm.at[wid])
    pltpu.sync_copy(idxl_v, idx_hbm.at[wid, 0])
    cnt_v[...] = jnp.full((L,), n, jnp.int32)
    pltpu.sync_copy(cnt_v, cnt_hbm.at[wid])


def _ln(z, g, be):
    mean = jnp.mean(z, axis=-1, keepdims=True)
    zc = z - mean
    var = jnp.mean(zc * zc, axis=-1, keepdims=True)
    return zc * lax.rsqrt(var + 1e-5) * g + be


def _tc_body(cnt_smem, x_ref, m_ref, idx_ref, xc_ref, wt_ref, b_ref, g_ref,
             be_ref, o_ref):
    w = pl.program_id(0)
    n = cnt_smem[w, 0]
    x = x_ref[...]

    @pl.when(n <= CAP)
    def _sparse():
        xc = xc_ref[0]
        soft = lax.dot_general(
            xc.astype(jnp.bfloat16), wt_ref[...],
            (((1,), (0,)), ((), ())), preferred_element_type=jnp.float32,
        ) + b_ref[...]
        normed = _ln(xc + soft, g_ref[...], be_ref[...])
        d = (normed - xc).astype(jnp.bfloat16)
        idx = idx_ref[0]
        rows = lax.broadcasted_iota(jnp.int32, (SEG, CAP), 0)
        kio = lax.broadcasted_iota(jnp.int32, (SEG, CAP), 1)
        p = ((rows == idx) & (kio < n)).astype(jnp.bfloat16)
        upd = lax.dot_general(
            p, d, (((1,), (0,)), ((), ())), preferred_element_type=jnp.float32,
        )
        o_ref[...] = x + upd

    @pl.when(n > CAP)
    def _dense():
        soft = lax.dot_general(
            x.astype(jnp.bfloat16), wt_ref[...],
            (((1,), (0,)), ((), ())), preferred_element_type=jnp.float32,
        ) + b_ref[...]
        normed = _ln(x + soft, g_ref[...], be_ref[...])
        o_ref[...] = jnp.where(m_ref[...] == 1, normed, x)


def kernel(batch_embeddings, position_mask, W, b, gamma, beta):
    B, S, Hh = batch_embeddings.shape
    N = B * S
    x = batch_embeddings.reshape(N, Hh)
    mflat = position_mask.reshape(N).astype(jnp.int32)
    m2d = mflat.reshape(N, 1)
    wt = W.T.astype(jnp.bfloat16)
    b2 = b.reshape(1, Hh)
    g2 = gamma.reshape(1, Hh)
    be2 = beta.reshape(1, Hh)

    sc = pl.kernel(
        _sc_compact_gather,
        out_type=(
            jax.ShapeDtypeStruct((NW, 1, CAP), jnp.int32),
            jax.ShapeDtypeStruct((NW, L), jnp.int32),
            jax.ShapeDtypeStruct((NW, CAP, Hh), jnp.float32),
        ),
        mesh=_SC_MESH,
        compiler_params=pltpu.CompilerParams(needs_layout_passes=False),
        scratch_types=[
            pltpu.VMEM((SEG,), jnp.int32),
            pltpu.VMEM((CAP,), jnp.int32),
            pltpu.VMEM((CAP,), jnp.int32),
            pltpu.VMEM((CAP, Hh), jnp.float32),
            pltpu.VMEM((L,), jnp.int32),
            pltpu.SemaphoreType.DMA,
        ],
    )
    idx, cnt, xc = sc(mflat, x)

    out = pl.pallas_call(
        _tc_body,
        grid=(NW,),
        in_specs=[
            pl.BlockSpec(memory_space=pltpu.SMEM),
            pl.BlockSpec((SEG, Hh), lambda i: (i, 0)),
            pl.BlockSpec((SEG, 1), lambda i: (i, 0)),
            pl.BlockSpec((1, 1, CAP), lambda i: (i, 0, 0)),
            pl.BlockSpec((1, CAP, Hh), lambda i: (i, 0, 0)),
            pl.BlockSpec((Hh, Hh), lambda i: (0, 0)),
            pl.BlockSpec((1, Hh), lambda i: (0, 0)),
            pl.BlockSpec((1, Hh), lambda i: (0, 0)),
            pl.BlockSpec((1, Hh), lambda i: (0, 0)),
        ],
        out_specs=pl.BlockSpec((SEG, Hh), lambda i: (i, 0)),
        out_shape=jax.ShapeDtypeStruct((N, Hh), jnp.float32),
        compiler_params=pltpu.CompilerParams(
            dimension_semantics=("arbitrary",),
        ),
    )(cnt, x, m2d, idx, xc, wt, b2, g2, be2)
    return out.reshape(B, S, Hh)


# speedup vs baseline: 1.0568x; 1.0568x over previous
import jax
import jax.numpy as jnp
from jax import lax
from jax.experimental import pallas as pl
from jax.experimental.pallas import tpu as pltpu
from jax.experimental.pallas import tpu_sc as plsc

H = 768
NW = 32
SEG = 1024
CAP = 128
L = 16
BLK = 2 * SEG

_SC_MESH = plsc.VectorSubcoreMesh(
    core_axis_name="c", subcore_axis_name="s", num_cores=2, num_subcores=16
)


def _sc_compact_gather(mask_hbm, x_hbm, idx_hbm, cnt_hbm, xc_hbm,
                       mask_v, idxl_v, idxg_v, rows_v, cnt_v, sem):
    wid = lax.axis_index("s") * 2 + lax.axis_index("c")
    base = wid * SEG
    pltpu.sync_copy(mask_hbm.at[pl.ds(base, SEG)], mask_v)

    zeros = jnp.zeros((L,), jnp.int32)
    for i in range(CAP // L):
        idxl_v[pl.ds(i * L, L)] = zeros

    lane = lax.iota(jnp.int32, L)

    def body(c, off_vec):
        mchunk = mask_v[pl.ds(c * L, L)]
        sel = mchunk == 1
        seli = jnp.where(sel, 1, 0).astype(jnp.int32)
        pos = off_vec + plsc.cumsum(seli) - 1
        okay = sel & (pos < CAP)
        posc = jnp.minimum(pos, CAP - 1)
        localpos = c * L + lane
        plsc.store_scatter(idxl_v, [posc], localpos, mask=okay)
        return off_vec + plsc.all_reduce_population_count(sel)

    n_vec = lax.fori_loop(0, SEG // L, body, jnp.zeros((L,), jnp.int32),
                          unroll=4)

    for i in range(CAP // L):
        idxg_v[pl.ds(i * L, L)] = idxl_v[pl.ds(i * L, L)] + base

    pltpu.async_copy(x_hbm.at[idxg_v], rows_v, sem).wait()
    pltpu.sync_copy(rows_v, xc_hbm.at[wid])
    pltpu.sync_copy(idxl_v, idx_hbm.at[wid, 0])
    cnt_v[...] = n_vec
    pltpu.sync_copy(cnt_v, cnt_hbm.at[wid])


def _ln(z, g, be):
    mean = jnp.mean(z, axis=-1, keepdims=True)
    zc = z - mean
    var = jnp.mean(zc * zc, axis=-1, keepdims=True)
    return zc * lax.rsqrt(var + 1e-5) * g + be


def _bf16_dot(a, bmat):
    return lax.dot_general(
        a, bmat, (((1,), (0,)), ((), ())), preferred_element_type=jnp.float32
    )


def _tc_body(cnt_smem, x_ref, m_ref, idx_ref, xc_ref, wt_ref, b_ref, g_ref,
             be_ref, o_ref):
    i = pl.program_id(0)
    n0 = cnt_smem[2 * i, 0]
    n1 = cnt_smem[2 * i + 1, 0]

    @pl.when(jnp.logical_and(n0 <= CAP, n1 <= CAP))
    def _sparse():
        xcb = xc_ref[...].reshape(2 * CAP, H)
        soft = _bf16_dot(xcb.astype(jnp.bfloat16), wt_ref[...]) + b_ref[...]
        normed = _ln(xcb + soft, g_ref[...], be_ref[...])
        d = (normed - xcb).astype(jnp.bfloat16)
        rows = lax.broadcasted_iota(jnp.int32, (SEG, CAP), 0)
        kio = lax.broadcasted_iota(jnp.int32, (SEG, CAP), 1)
        p0 = ((rows == idx_ref[0]) & (kio < n0)).astype(jnp.bfloat16)
        p1 = ((rows == idx_ref[1]) & (kio < n1)).astype(jnp.bfloat16)
        o_ref[0:SEG, :] = x_ref[0:SEG, :] + _bf16_dot(p0, d[0:CAP])
        o_ref[SEG:BLK, :] = x_ref[SEG:BLK, :] + _bf16_dot(p1, d[CAP:2 * CAP])

    @pl.when(jnp.logical_or(n0 > CAP, n1 > CAP))
    def _dense():
        x = x_ref[...]
        soft = _bf16_dot(x.astype(jnp.bfloat16), wt_ref[...]) + b_ref[...]
        normed = _ln(x + soft, g_ref[...], be_ref[...])
        o_ref[...] = jnp.where(m_ref[...] == 1, normed, x)


def kernel(batch_embeddings, position_mask, W, b, gamma, beta):
    B, S, Hh = batch_embeddings.shape
    N = B * S
    x = batch_embeddings.reshape(N, Hh)
    mflat = position_mask.reshape(N).astype(jnp.int32)
    m2d = mflat.reshape(N, 1)
    wt = W.T.astype(jnp.bfloat16)
    b2 = b.reshape(1, Hh)
    g2 = gamma.reshape(1, Hh)
    be2 = beta.reshape(1, Hh)

    sc = pl.kernel(
        _sc_compact_gather,
        out_type=(
            jax.ShapeDtypeStruct((NW, 1, CAP), jnp.int32),
            jax.ShapeDtypeStruct((NW, L), jnp.int32),
            jax.ShapeDtypeStruct((NW, CAP, Hh), jnp.float32),
        ),
        mesh=_SC_MESH,
        compiler_params=pltpu.CompilerParams(needs_layout_passes=False),
        scratch_types=[
            pltpu.VMEM((SEG,), jnp.int32),
            pltpu.VMEM((CAP,), jnp.int32),
            pltpu.VMEM((CAP,), jnp.int32),
            pltpu.VMEM((CAP, Hh), jnp.float32),
            pltpu.VMEM((L,), jnp.int32),
            pltpu.SemaphoreType.DMA,
        ],
    )
    idx, cnt, xc = sc(mflat, x)

    out = pl.pallas_call(
        _tc_body,
        grid=(N // BLK,),
        in_specs=[
            pl.BlockSpec(memory_space=pltpu.SMEM),
            pl.BlockSpec((BLK, Hh), lambda i: (i, 0)),
            pl.BlockSpec((BLK, 1), lambda i: (i, 0)),
            pl.BlockSpec((2, 1, CAP), lambda i: (i, 0, 0)),
            pl.BlockSpec((2, CAP, Hh), lambda i: (i, 0, 0)),
            pl.BlockSpec((Hh, Hh), lambda i: (0, 0)),
            pl.BlockSpec((1, Hh), lambda i: (0, 0)),
            pl.BlockSpec((1, Hh), lambda i: (0, 0)),
            pl.BlockSpec((1, Hh), lambda i: (0, 0)),
        ],
        out_specs=pl.BlockSpec((BLK, Hh), lambda i: (i, 0)),
        out_shape=jax.ShapeDtypeStruct((N, Hh), jnp.float32),
        compiler_params=pltpu.CompilerParams(
            dimension_semantics=("arbitrary",),
        ),
    )(cnt, x, m2d, idx, xc, wt, b2, g2, be2)
    return out.reshape(B, S, Hh)


# speedup vs baseline: 1.1374x; 1.0763x over previous
import jax
import jax.numpy as jnp
from jax import lax
from jax.experimental import pallas as pl
from jax.experimental.pallas import tpu as pltpu
from jax.experimental.pallas import tpu_sc as plsc

H = 768
NW = 32
SEG = 1024
CAP = 128
L = 16
BLK = 2 * SEG


_SC_MESH = plsc.VectorSubcoreMesh(
    core_axis_name="c", subcore_axis_name="s", num_cores=2, num_subcores=16
)


def _sc_compact_gather(mask_hbm, x_hbm, idx_hbm, cnt_hbm, xc_hbm,
                       mask_v, idxl_v, idxg_v, rows_v, cnt_v, sem):
    wid = lax.axis_index("s") * 2 + lax.axis_index("c")
    base = wid * SEG
    pltpu.sync_copy(mask_hbm.at[pl.ds(base, SEG)], mask_v)

    zeros = jnp.zeros((L,), jnp.int32)
    for i in range(CAP // L):
        idxl_v[pl.ds(i * L, L)] = zeros

    lane = lax.iota(jnp.int32, L)

    def body(c, off_vec):
        mchunk = mask_v[pl.ds(c * L, L)]
        sel = mchunk == 1
        seli = jnp.where(sel, 1, 0).astype(jnp.int32)
        pos = off_vec + plsc.cumsum(seli) - 1
        okay = sel & (pos < CAP)
        posc = jnp.minimum(pos, CAP - 1)
        localpos = c * L + lane
        plsc.store_scatter(idxl_v, [posc], localpos, mask=okay)
        return off_vec + plsc.all_reduce_population_count(sel)

    n_vec = lax.fori_loop(0, SEG // L, body, jnp.zeros((L,), jnp.int32),
                          unroll=4)

    for i in range(CAP // L):
        idxg_v[pl.ds(i * L, L)] = idxl_v[pl.ds(i * L, L)] + base

    pltpu.async_copy(x_hbm.at[idxg_v], rows_v, sem).wait()
    pltpu.sync_copy(rows_v, xc_hbm.at[wid])
    pltpu.sync_copy(idxl_v, idx_hbm.at[wid, 0])
    cnt_v[...] = n_vec
    pltpu.sync_copy(cnt_v, cnt_hbm.at[wid])


def _ln(z, g, be):
    mean = jnp.mean(z, axis=-1, keepdims=True)
    zc = z - mean
    var = jnp.mean(zc * zc, axis=-1, keepdims=True)
    return zc * lax.rsqrt(var + 1e-5) * g + be


def _bf16_dot(a, bmat):
    return lax.dot_general(
        a, bmat, (((1,), (0,)), ((), ())), preferred_element_type=jnp.float32
    )


def _sparse_body(cnt_smem, x_ref, idx_ref, xc_ref, wt_ref, b_ref, g_ref,
                 be_ref, o_ref):
    i = pl.program_id(0)
    n0 = cnt_smem[2 * i, 0]
    n1 = cnt_smem[2 * i + 1, 0]
    xcb = xc_ref[...].reshape(2 * CAP, H)
    soft = _bf16_dot(xcb.astype(jnp.bfloat16), wt_ref[...]) + b_ref[...]
    normed = _ln(xcb + soft, g_ref[...], be_ref[...])
    d = (normed - xcb).astype(jnp.bfloat16)
    rows = lax.broadcasted_iota(jnp.int32, (SEG, CAP), 0)
    kio = lax.broadcasted_iota(jnp.int32, (SEG, CAP), 1)
    p0 = ((rows == idx_ref[0]) & (kio < n0)).astype(jnp.bfloat16)
    p1 = ((rows == idx_ref[1]) & (kio < n1)).astype(jnp.bfloat16)
    o_ref[0:SEG, :] = x_ref[0:SEG, :] + _bf16_dot(p0, d[0:CAP])
    o_ref[SEG:BLK, :] = x_ref[SEG:BLK, :] + _bf16_dot(p1, d[CAP:2 * CAP])


def _dense_body(x_ref, m_ref, wt_ref, b_ref, g_ref, be_ref, o_ref):
    x = x_ref[...]
    soft = _bf16_dot(x.astype(jnp.bfloat16), wt_ref[...]) + b_ref[...]
    normed = _ln(x + soft, g_ref[...], be_ref[...])
    o_ref[...] = jnp.where(m_ref[...] == 1, normed, x)


def kernel(batch_embeddings, position_mask, W, b, gamma, beta):
    B, S, Hh = batch_embeddings.shape
    N = B * S
    x = batch_embeddings.reshape(N, Hh)
    mflat = position_mask.reshape(N).astype(jnp.int32)
    wt = W.T.astype(jnp.bfloat16)
    b2 = b.reshape(1, Hh)
    g2 = gamma.reshape(1, Hh)
    be2 = beta.reshape(1, Hh)

    sc = pl.kernel(
        _sc_compact_gather,
        out_type=(
            jax.ShapeDtypeStruct((NW, 1, CAP), jnp.int32),
            jax.ShapeDtypeStruct((NW, L), jnp.int32),
            jax.ShapeDtypeStruct((NW, CAP, Hh), jnp.float32),
        ),
        mesh=_SC_MESH,
        compiler_params=pltpu.CompilerParams(needs_layout_passes=False),
        scratch_types=[
            pltpu.VMEM((SEG,), jnp.int32),
            pltpu.VMEM((CAP,), jnp.int32),
            pltpu.VMEM((CAP,), jnp.int32),
            pltpu.VMEM((CAP, Hh), jnp.float32),
            pltpu.VMEM((L,), jnp.int32),
            pltpu.SemaphoreType.DMA,
        ],
    )
    idx, cnt, xc = sc(mflat, x)

    def sparse_path(ops):
        xx, cc, ii, xcc = ops
        return pl.pallas_call(
            _sparse_body,
            grid=(N // BLK,),
            in_specs=[
                pl.BlockSpec(memory_space=pltpu.SMEM),
                pl.BlockSpec((BLK, Hh), lambda i: (i, 0)),
                pl.BlockSpec((2, 1, CAP), lambda i: (i, 0, 0)),
                pl.BlockSpec((2, CAP, Hh), lambda i: (i, 0, 0)),
                pl.BlockSpec((Hh, Hh), lambda i: (0, 0)),
                pl.BlockSpec((1, Hh), lambda i: (0, 0)),
                pl.BlockSpec((1, Hh), lambda i: (0, 0)),
                pl.BlockSpec((1, Hh), lambda i: (0, 0)),
            ],
            out_specs=pl.BlockSpec((BLK, Hh), lambda i: (i, 0)),
            out_shape=jax.ShapeDtypeStruct((N, Hh), jnp.float32),
            compiler_params=pltpu.CompilerParams(
                dimension_semantics=("arbitrary",),
            ),
        )(cc, xx, ii, xcc, wt, b2, g2, be2)

    def dense_path(ops):
        xx, cc, ii, xcc = ops
        return pl.pallas_call(
            _dense_body,
            grid=(N // BLK,),
            in_specs=[
                pl.BlockSpec((BLK, Hh), lambda i: (i, 0)),
                pl.BlockSpec((BLK, 1), lambda i: (i, 0)),
                pl.BlockSpec((Hh, Hh), lambda i: (0, 0)),
                pl.BlockSpec((1, Hh), lambda i: (0, 0)),
                pl.BlockSpec((1, Hh), lambda i: (0, 0)),
                pl.BlockSpec((1, Hh), lambda i: (0, 0)),
            ],
            out_specs=pl.BlockSpec((BLK, Hh), lambda i: (i, 0)),
            out_shape=jax.ShapeDtypeStruct((N, Hh), jnp.float32),
            compiler_params=pltpu.CompilerParams(
                dimension_semantics=("arbitrary",),
            ),
        )(xx, mflat.reshape(N, 1), wt, b2, g2, be2)

    overflow = jnp.any(cnt[:, 0] > CAP)
    out = lax.cond(overflow, dense_path, sparse_path, (x, cnt, idx, xc))
    return out.reshape(B, S, Hh)


# speedup vs baseline: 1.1376x; 1.0002x over previous
import jax
import jax.numpy as jnp
from jax import lax
from jax.experimental import pallas as pl
from jax.experimental.pallas import tpu as pltpu
from jax.experimental.pallas import tpu_sc as plsc

H = 768
NW = 32
SEG = 1024
CAP = 128
L = 16
BLK = 2 * SEG
SPB = BLK // SEG


_SC_MESH = plsc.VectorSubcoreMesh(
    core_axis_name="c", subcore_axis_name="s", num_cores=2, num_subcores=16
)


def _sc_compact_gather(mask_hbm, x_hbm, idx_hbm, cnt_hbm, xc_hbm,
                       mask_v, idxl_v, idxg_v, rows_v, cnt_v, sem):
    wid = lax.axis_index("s") * 2 + lax.axis_index("c")
    base = wid * SEG
    pltpu.sync_copy(mask_hbm.at[pl.ds(base, SEG)], mask_v)

    zeros = jnp.zeros((L,), jnp.int32)
    for i in range(CAP // L):
        idxl_v[pl.ds(i * L, L)] = zeros

    lane = lax.iota(jnp.int32, L)

    def body(c, off_vec):
        mchunk = mask_v[pl.ds(c * L, L)]
        sel = mchunk == 1
        seli = jnp.where(sel, 1, 0).astype(jnp.int32)
        pos = off_vec + plsc.cumsum(seli) - 1
        okay = sel & (pos < CAP)
        posc = jnp.minimum(pos, CAP - 1)
        localpos = c * L + lane
        plsc.store_scatter(idxl_v, [posc], localpos, mask=okay)
        return off_vec + plsc.all_reduce_population_count(sel)

    n_vec = lax.fori_loop(0, SEG // L, body, jnp.zeros((L,), jnp.int32),
                          unroll=4)

    for i in range(CAP // L):
        idxg_v[pl.ds(i * L, L)] = idxl_v[pl.ds(i * L, L)] + base

    pltpu.async_copy(x_hbm.at[idxg_v], rows_v, sem).wait()
    pltpu.sync_copy(rows_v, xc_hbm.at[wid])
    pltpu.sync_copy(idxl_v, idx_hbm.at[wid, 0])
    cnt_v[...] = n_vec
    pltpu.sync_copy(cnt_v, cnt_hbm.at[wid])


def _ln(z, g, be):
    mean = jnp.mean(z, axis=-1, keepdims=True)
    zc = z - mean
    var = jnp.mean(zc * zc, axis=-1, keepdims=True)
    return zc * lax.rsqrt(var + 1e-5) * g + be


def _bf16_dot(a, bmat):
    return lax.dot_general(
        a, bmat, (((1,), (0,)), ((), ())), preferred_element_type=jnp.float32
    )


def _sparse_body(cnt_smem, x_ref, idx_ref, xc_ref, wt_ref, b_ref, g_ref,
                 be_ref, o_ref):
    i = pl.program_id(0)
    xcb = xc_ref[...].reshape(SPB * CAP, H)
    soft = _bf16_dot(xcb.astype(jnp.bfloat16), wt_ref[...]) + b_ref[...]
    normed = _ln(xcb + soft, g_ref[...], be_ref[...])
    d = (normed - xcb).astype(jnp.bfloat16)
    rows = lax.broadcasted_iota(jnp.int32, (SEG, CAP), 0)
    kio = lax.broadcasted_iota(jnp.int32, (SEG, CAP), 1)
    for s in range(SPB):
        n = cnt_smem[SPB * i + s, 0]
        p = ((rows == idx_ref[s]) & (kio < n)).astype(jnp.bfloat16)
        o_ref[s * SEG:(s + 1) * SEG, :] = (
            x_ref[s * SEG:(s + 1) * SEG, :]
            + _bf16_dot(p, d[s * CAP:(s + 1) * CAP])
        )


def _dense_body(x_ref, m_ref, wt_ref, b_ref, g_ref, be_ref, o_ref):
    x = x_ref[...]
    soft = _bf16_dot(x.astype(jnp.bfloat16), wt_ref[...]) + b_ref[...]
    normed = _ln(x + soft, g_ref[...], be_ref[...])
    o_ref[...] = jnp.where(m_ref[...] == 1, normed, x)


def kernel(batch_embeddings, position_mask, W, b, gamma, beta):
    B, S, Hh = batch_embeddings.shape
    N = B * S
    x = batch_embeddings.reshape(N, Hh)
    mflat = position_mask.reshape(N).astype(jnp.int32)
    wt = W.T.astype(jnp.bfloat16)
    b2 = b.reshape(1, Hh)
    g2 = gamma.reshape(1, Hh)
    be2 = beta.reshape(1, Hh)

    sc = pl.kernel(
        _sc_compact_gather,
        out_type=(
            jax.ShapeDtypeStruct((NW, 1, CAP), jnp.int32),
            jax.ShapeDtypeStruct((NW, L), jnp.int32),
            jax.ShapeDtypeStruct((NW, CAP, Hh), jnp.float32),
        ),
        mesh=_SC_MESH,
        compiler_params=pltpu.CompilerParams(needs_layout_passes=False),
        scratch_types=[
            pltpu.VMEM((SEG,), jnp.int32),
            pltpu.VMEM((CAP,), jnp.int32),
            pltpu.VMEM((CAP,), jnp.int32),
            pltpu.VMEM((CAP, Hh), jnp.float32),
            pltpu.VMEM((L,), jnp.int32),
            pltpu.SemaphoreType.DMA,
        ],
    )
    idx, cnt, xc = sc(mflat, x)

    def sparse_path(ops):
        xx, cc, ii, xcc = ops
        return pl.pallas_call(
            _sparse_body,
            grid=(N // BLK,),
            in_specs=[
                pl.BlockSpec(memory_space=pltpu.SMEM),
                pl.BlockSpec((BLK, Hh), lambda i: (i, 0)),
                pl.BlockSpec((SPB, 1, CAP), lambda i: (i, 0, 0)),
                pl.BlockSpec((SPB, CAP, Hh), lambda i: (i, 0, 0)),
                pl.BlockSpec((Hh, Hh), lambda i: (0, 0)),
                pl.BlockSpec((1, Hh), lambda i: (0, 0)),
                pl.BlockSpec((1, Hh), lambda i: (0, 0)),
                pl.BlockSpec((1, Hh), lambda i: (0, 0)),
            ],
            out_specs=pl.BlockSpec((BLK, Hh), lambda i: (i, 0)),
            out_shape=jax.ShapeDtypeStruct((N, Hh), jnp.float32),
            compiler_params=pltpu.CompilerParams(
                dimension_semantics=("arbitrary",),
            ),
        )(cc, xx, ii, xcc, wt, b2, g2, be2)

    def dense_path(ops):
        xx, cc, ii, xcc = ops
        return pl.pallas_call(
            _dense_body,
            grid=(N // BLK,),
            in_specs=[
                pl.BlockSpec((BLK, Hh), lambda i: (i, 0)),
                pl.BlockSpec((BLK, 1), lambda i: (i, 0)),
                pl.BlockSpec((Hh, Hh), lambda i: (0, 0)),
                pl.BlockSpec((1, Hh), lambda i: (0, 0)),
                pl.BlockSpec((1, Hh), lambda i: (0, 0)),
                pl.BlockSpec((1, Hh), lambda i: (0, 0)),
            ],
            out_specs=pl.BlockSpec((BLK, Hh), lambda i: (i, 0)),
            out_shape=jax.ShapeDtypeStruct((N, Hh), jnp.float32),
            compiler_params=pltpu.CompilerParams(
                dimension_semantics=("arbitrary",),
            ),
        )(xx, mflat.reshape(N, 1), wt, b2, g2, be2)

    overflow = jnp.any(cnt[:, 0] > CAP)
    out = lax.cond(overflow, dense_path, sparse_path, (x, cnt, idx, xc))
    return out.reshape(B, S, Hh)


# speedup vs baseline: 1.1519x; 1.0126x over previous
import jax
import jax.numpy as jnp
from jax import lax
from jax.experimental import pallas as pl
from jax.experimental.pallas import tpu as pltpu
from jax.experimental.pallas import tpu_sc as plsc

H = 768
NW = 32
SEG = 1024
CAP = 128
L = 16
BLK = 2 * SEG
SPB = BLK // SEG


_SC_MESH = plsc.VectorSubcoreMesh(
    core_axis_name="c", subcore_axis_name="s", num_cores=2, num_subcores=16
)


NCH = 4
CH = CAP // NCH


def _sc_compact_gather(mask_hbm, x_hbm, idx_hbm, cnt_hbm, xc_hbm,
                       mask_v, idxl_v, idxg0, idxg1, idxg2, idxg3,
                       rows_v, cnt_v, gsem, wsem):
    wid = lax.axis_index("s") * 2 + lax.axis_index("c")
    base = wid * SEG
    pltpu.sync_copy(mask_hbm.at[pl.ds(base, SEG)], mask_v)

    zeros = jnp.zeros((L,), jnp.int32)
    for i in range(CAP // L):
        idxl_v[pl.ds(i * L, L)] = zeros

    lane = lax.iota(jnp.int32, L)

    def body(c, off_vec):
        mchunk = mask_v[pl.ds(c * L, L)]
        sel = mchunk == 1
        seli = jnp.where(sel, 1, 0).astype(jnp.int32)
        pos = off_vec + plsc.cumsum(seli) - 1
        okay = sel & (pos < CAP)
        posc = jnp.minimum(pos, CAP - 1)
        localpos = c * L + lane
        plsc.store_scatter(idxl_v, [posc], localpos, mask=okay)
        return off_vec + plsc.all_reduce_population_count(sel)

    n_vec = lax.fori_loop(0, SEG // L, body, jnp.zeros((L,), jnp.int32),
                          unroll=4)

    idxgs = (idxg0, idxg1, idxg2, idxg3)
    for c in range(NCH):
        for i in range(CH // L):
            idxgs[c][pl.ds(i * L, L)] = (
                idxl_v[pl.ds(c * CH + i * L, L)] + base
            )

    gets = [
        pltpu.async_copy(x_hbm.at[idxgs[c]], rows_v.at[pl.ds(c * CH, CH)],
                         gsem)
        for c in range(NCH)
    ]
    puts = []
    for c in range(NCH):
        gets[c].wait()
        puts.append(
            pltpu.async_copy(rows_v.at[pl.ds(c * CH, CH)],
                             xc_hbm.at[wid, pl.ds(c * CH, CH)], wsem)
        )
    pltpu.sync_copy(idxl_v, idx_hbm.at[wid, 0])
    cnt_v[...] = n_vec
    pltpu.sync_copy(cnt_v, cnt_hbm.at[wid])
    for p in puts:
        p.wait()


def _ln(z, g, be):
    mean = jnp.mean(z, axis=-1, keepdims=True)
    zc = z - mean
    var = jnp.mean(zc * zc, axis=-1, keepdims=True)
    return zc * lax.rsqrt(var + 1e-5) * g + be


def _bf16_dot(a, bmat):
    return lax.dot_general(
        a, bmat, (((1,), (0,)), ((), ())), preferred_element_type=jnp.float32
    )


def _sparse_body(cnt_smem, x_ref, idx_ref, xc_ref, wt_ref, b_ref, g_ref,
                 be_ref, o_ref):
    i = pl.program_id(0)
    xcb = xc_ref[...].reshape(SPB * CAP, H)
    soft = _bf16_dot(xcb.astype(jnp.bfloat16), wt_ref[...]) + b_ref[...]
    normed = _ln(xcb + soft, g_ref[...], be_ref[...])
    d = (normed - xcb).astype(jnp.bfloat16)
    rows = lax.broadcasted_iota(jnp.int32, (SEG, CAP), 0)
    kio = lax.broadcasted_iota(jnp.int32, (SEG, CAP), 1)
    for s in range(SPB):
        n = cnt_smem[SPB * i + s, 0]
        p = ((rows == idx_ref[s]) & (kio < n)).astype(jnp.bfloat16)
        o_ref[s * SEG:(s + 1) * SEG, :] = (
            x_ref[s * SEG:(s + 1) * SEG, :]
            + _bf16_dot(p, d[s * CAP:(s + 1) * CAP])
        )


def _dense_body(x_ref, m_ref, wt_ref, b_ref, g_ref, be_ref, o_ref):
    x = x_ref[...]
    soft = _bf16_dot(x.astype(jnp.bfloat16), wt_ref[...]) + b_ref[...]
    normed = _ln(x + soft, g_ref[...], be_ref[...])
    o_ref[...] = jnp.where(m_ref[...] == 1, normed, x)


def kernel(batch_embeddings, position_mask, W, b, gamma, beta):
    B, S, Hh = batch_embeddings.shape
    N = B * S
    x = batch_embeddings.reshape(N, Hh)
    mflat = position_mask.reshape(N).astype(jnp.int32)
    wt = W.T.astype(jnp.bfloat16)
    b2 = b.reshape(1, Hh)
    g2 = gamma.reshape(1, Hh)
    be2 = beta.reshape(1, Hh)

    sc = pl.kernel(
        _sc_compact_gather,
        out_type=(
            jax.ShapeDtypeStruct((NW, 1, CAP), jnp.int32),
            jax.ShapeDtypeStruct((NW, L), jnp.int32),
            jax.ShapeDtypeStruct((NW, CAP, Hh), jnp.float32),
        ),
        mesh=_SC_MESH,
        compiler_params=pltpu.CompilerParams(needs_layout_passes=False),
        scratch_types=[
            pltpu.VMEM((SEG,), jnp.int32),
            pltpu.VMEM((CAP,), jnp.int32),
            pltpu.VMEM((CH,), jnp.int32),
            pltpu.VMEM((CH,), jnp.int32),
            pltpu.VMEM((CH,), jnp.int32),
            pltpu.VMEM((CH,), jnp.int32),
            pltpu.VMEM((CAP, Hh), jnp.float32),
            pltpu.VMEM((L,), jnp.int32),
            pltpu.SemaphoreType.DMA,
            pltpu.SemaphoreType.DMA,
        ],
    )
    idx, cnt, xc = sc(mflat, x)

    def sparse_path(ops):
        xx, cc, ii, xcc = ops
        return pl.pallas_call(
            _sparse_body,
            grid=(N // BLK,),
            in_specs=[
                pl.BlockSpec(memory_space=pltpu.SMEM),
                pl.BlockSpec((BLK, Hh), lambda i: (i, 0)),
                pl.BlockSpec((SPB, 1, CAP), lambda i: (i, 0, 0)),
                pl.BlockSpec((SPB, CAP, Hh), lambda i: (i, 0, 0)),
                pl.BlockSpec((Hh, Hh), lambda i: (0, 0)),
                pl.BlockSpec((1, Hh), lambda i: (0, 0)),
                pl.BlockSpec((1, Hh), lambda i: (0, 0)),
                pl.BlockSpec((1, Hh), lambda i: (0, 0)),
            ],
            out_specs=pl.BlockSpec((BLK, Hh), lambda i: (i, 0)),
            out_shape=jax.ShapeDtypeStruct((N, Hh), jnp.float32),
            compiler_params=pltpu.CompilerParams(
                dimension_semantics=("arbitrary",),
            ),
        )(cc, xx, ii, xcc, wt, b2, g2, be2)

    def dense_path(ops):
        xx, cc, ii, xcc = ops
        return pl.pallas_call(
            _dense_body,
            grid=(N // BLK,),
            in_specs=[
                pl.BlockSpec((BLK, Hh), lambda i: (i, 0)),
                pl.BlockSpec((BLK, 1), lambda i: (i, 0)),
                pl.BlockSpec((Hh, Hh), lambda i: (0, 0)),
                pl.BlockSpec((1, Hh), lambda i: (0, 0)),
                pl.BlockSpec((1, Hh), lambda i: (0, 0)),
                pl.BlockSpec((1, Hh), lambda i: (0, 0)),
            ],
            out_specs=pl.BlockSpec((BLK, Hh), lambda i: (i, 0)),
            out_shape=jax.ShapeDtypeStruct((N, Hh), jnp.float32),
            compiler_params=pltpu.CompilerParams(
                dimension_semantics=("arbitrary",),
            ),
        )(xx, mflat.reshape(N, 1), wt, b2, g2, be2)

    overflow = jnp.any(cnt[:, 0] > CAP)
    out = lax.cond(overflow, dense_path, sparse_path, (x, cnt, idx, xc))
    return out.reshape(B, S, Hh)


# speedup vs baseline: 1.4435x; 1.2531x over previous
import jax
import jax.numpy as jnp
from jax import lax
from jax.experimental import pallas as pl
from jax.experimental.pallas import tpu as pltpu
from jax.experimental.pallas import tpu_sc as plsc

H = 768
NW = 32
SEG = 1024
CAP = 128
L = 16
BLK = 2 * SEG
SPB = BLK // SEG


_SC_MESH = plsc.VectorSubcoreMesh(
    core_axis_name="c", subcore_axis_name="s", num_cores=2, num_subcores=16
)


def _sc_compact(mask_hbm, idx_hbm, cnt_hbm, mask_v, idxl_v, cnt_v):
    wid = lax.axis_index("s") * 2 + lax.axis_index("c")
    base = wid * SEG
    pltpu.sync_copy(mask_hbm.at[pl.ds(base, SEG)], mask_v)

    zeros = jnp.zeros((L,), jnp.int32)
    for i in range(CAP // L):
        idxl_v[pl.ds(i * L, L)] = zeros

    lane = lax.iota(jnp.int32, L)

    def body(c, off_vec):
        mchunk = mask_v[pl.ds(c * L, L)]
        sel = mchunk == 1
        seli = jnp.where(sel, 1, 0).astype(jnp.int32)
        pos = off_vec + plsc.cumsum(seli) - 1
        okay = sel & (pos < CAP)
        posc = jnp.minimum(pos, CAP - 1)
        localpos = c * L + lane
        plsc.store_scatter(idxl_v, [posc], localpos, mask=okay)
        return off_vec + plsc.all_reduce_population_count(sel)

    n_vec = lax.fori_loop(0, SEG // L, body, jnp.zeros((L,), jnp.int32),
                          unroll=4)

    pltpu.sync_copy(idxl_v, idx_hbm.at[wid, 0])
    cnt_v[...] = n_vec
    pltpu.sync_copy(cnt_v, cnt_hbm.at[wid])


def _ln(z, g, be):
    mean = jnp.mean(z, axis=-1, keepdims=True)
    zc = z - mean
    var = jnp.mean(zc * zc, axis=-1, keepdims=True)
    return zc * lax.rsqrt(var + 1e-5) * g + be


def _bf16_dot(a, bmat):
    return lax.dot_general(
        a, bmat, (((1,), (0,)), ((), ())), preferred_element_type=jnp.float32
    )


def _sparse_body(cnt_smem, x_ref, idx_ref, wt_ref, b_ref, g_ref,
                 be_ref, o_ref):
    i = pl.program_id(0)
    rows = lax.broadcasted_iota(jnp.int32, (SEG, CAP), 0)
    kio = lax.broadcasted_iota(jnp.int32, (SEG, CAP), 1)
    cols = lax.broadcasted_iota(jnp.int32, (CAP, SEG), 1)
    krow = lax.broadcasted_iota(jnp.int32, (CAP, SEG), 0)
    xgs = []
    ps = []
    for s in range(SPB):
        n = cnt_smem[SPB * i + s, 0]
        idxr = idx_ref[s]
        idxc = idxr.reshape(CAP, 1)
        gmat = ((cols == idxc) & (krow < n)).astype(jnp.bfloat16)
        xb = x_ref[s * SEG:(s + 1) * SEG, :].astype(jnp.bfloat16)
        xgs.append(_bf16_dot(gmat, xb))
        ps.append(((rows == idxr) & (kio < n)).astype(jnp.bfloat16))
    xg = jnp.concatenate(xgs, axis=0)
    soft = _bf16_dot(xg.astype(jnp.bfloat16), wt_ref[...]) + b_ref[...]
    normed = _ln(xg + soft, g_ref[...], be_ref[...])
    d = (normed - xg).astype(jnp.bfloat16)
    for s in range(SPB):
        o_ref[s * SEG:(s + 1) * SEG, :] = (
            x_ref[s * SEG:(s + 1) * SEG, :]
            + _bf16_dot(ps[s], d[s * CAP:(s + 1) * CAP])
        )


def _dense_body(x_ref, m_ref, wt_ref, b_ref, g_ref, be_ref, o_ref):
    x = x_ref[...]
    soft = _bf16_dot(x.astype(jnp.bfloat16), wt_ref[...]) + b_ref[...]
    normed = _ln(x + soft, g_ref[...], be_ref[...])
    o_ref[...] = jnp.where(m_ref[...] == 1, normed, x)


def kernel(batch_embeddings, position_mask, W, b, gamma, beta):
    B, S, Hh = batch_embeddings.shape
    N = B * S
    x = batch_embeddings.reshape(N, Hh)
    mflat = position_mask.reshape(N).astype(jnp.int32)
    wt = W.T.astype(jnp.bfloat16)
    b2 = b.reshape(1, Hh)
    g2 = gamma.reshape(1, Hh)
    be2 = beta.reshape(1, Hh)

    sc = pl.kernel(
        _sc_compact,
        out_type=(
            jax.ShapeDtypeStruct((NW, 1, CAP), jnp.int32),
            jax.ShapeDtypeStruct((NW, L), jnp.int32),
        ),
        mesh=_SC_MESH,
        compiler_params=pltpu.CompilerParams(needs_layout_passes=False),
        scratch_types=[
            pltpu.VMEM((SEG,), jnp.int32),
            pltpu.VMEM((CAP,), jnp.int32),
            pltpu.VMEM((L,), jnp.int32),
        ],
    )
    idx, cnt = sc(mflat)

    def sparse_path(ops):
        xx, cc, ii = ops
        return pl.pallas_call(
            _sparse_body,
            grid=(N // BLK,),
            in_specs=[
                pl.BlockSpec(memory_space=pltpu.SMEM),
                pl.BlockSpec((BLK, Hh), lambda i: (i, 0)),
                pl.BlockSpec((SPB, 1, CAP), lambda i: (i, 0, 0)),
                pl.BlockSpec((Hh, Hh), lambda i: (0, 0)),
                pl.BlockSpec((1, Hh), lambda i: (0, 0)),
                pl.BlockSpec((1, Hh), lambda i: (0, 0)),
                pl.BlockSpec((1, Hh), lambda i: (0, 0)),
            ],
            out_specs=pl.BlockSpec((BLK, Hh), lambda i: (i, 0)),
            out_shape=jax.ShapeDtypeStruct((N, Hh), jnp.float32),
            compiler_params=pltpu.CompilerParams(
                dimension_semantics=("arbitrary",),
            ),
        )(cc, xx, ii, wt, b2, g2, be2)

    def dense_path(ops):
        xx, cc, ii = ops
        return pl.pallas_call(
            _dense_body,
            grid=(N // BLK,),
            in_specs=[
                pl.BlockSpec((BLK, Hh), lambda i: (i, 0)),
                pl.BlockSpec((BLK, 1), lambda i: (i, 0)),
                pl.BlockSpec((Hh, Hh), lambda i: (0, 0)),
                pl.BlockSpec((1, Hh), lambda i: (0, 0)),
                pl.BlockSpec((1, Hh), lambda i: (0, 0)),
                pl.BlockSpec((1, Hh), lambda i: (0, 0)),
            ],
            out_specs=pl.BlockSpec((BLK, Hh), lambda i: (i, 0)),
            out_shape=jax.ShapeDtypeStruct((N, Hh), jnp.float32),
            compiler_params=pltpu.CompilerParams(
                dimension_semantics=("arbitrary",),
            ),
        )(xx, mflat.reshape(N, 1), wt, b2, g2, be2)

    overflow = jnp.any(cnt[:, 0] > CAP)
    out = lax.cond(overflow, dense_path, sparse_path, (x, cnt, idx))
    return out.reshape(B, S, Hh)


# speedup vs baseline: 1.4451x; 1.0011x over previous
import jax
import jax.numpy as jnp
from jax import lax
from jax.experimental import pallas as pl
from jax.experimental.pallas import tpu as pltpu
from jax.experimental.pallas import tpu_sc as plsc

H = 768
NW = 32
SEG = 1024
CAP = 128
L = 16
BLK = 2 * SEG
SPB = BLK // SEG


_SC_MESH = plsc.VectorSubcoreMesh(
    core_axis_name="c", subcore_axis_name="s", num_cores=2, num_subcores=16
)


def _sc_compact(mask_hbm, idx_hbm, cnt_hbm, mask_v, idxl_v, cnt_v):
    wid = lax.axis_index("s") * 2 + lax.axis_index("c")
    base = wid * SEG
    pltpu.sync_copy(mask_hbm.at[pl.ds(base, SEG)], mask_v)

    zeros = jnp.zeros((L,), jnp.int32)
    for i in range(CAP // L):
        idxl_v[pl.ds(i * L, L)] = zeros

    lane = lax.iota(jnp.int32, L)

    def body(c, off_vec):
        mchunk = mask_v[pl.ds(c * L, L)]
        sel = mchunk == 1
        seli = jnp.where(sel, 1, 0).astype(jnp.int32)
        pos = off_vec + plsc.cumsum(seli) - 1
        okay = sel & (pos < CAP)
        posc = jnp.minimum(pos, CAP - 1)
        localpos = c * L + lane
        plsc.store_scatter(idxl_v, [posc], localpos, mask=okay)
        return off_vec + plsc.all_reduce_population_count(sel)

    n_vec = lax.fori_loop(0, SEG // L, body, jnp.zeros((L,), jnp.int32),
                          unroll=4)

    pltpu.sync_copy(idxl_v, idx_hbm.at[wid, 0])
    cnt_v[...] = n_vec
    pltpu.sync_copy(cnt_v, cnt_hbm.at[wid])


def _ln(z, g, be):
    mean = jnp.mean(z, axis=-1, keepdims=True)
    zc = z - mean
    var = jnp.mean(zc * zc, axis=-1, keepdims=True)
    return zc * lax.rsqrt(var + 1e-5) * g + be


def _bf16_dot(a, bmat):
    return lax.dot_general(
        a, bmat, (((1,), (0,)), ((), ())), preferred_element_type=jnp.float32
    )


def _sparse_body(cnt_smem, x_ref, idx_ref, wt_ref, b_ref, g_ref,
                 be_ref, o_ref):
    i = pl.program_id(0)
    rows = lax.broadcasted_iota(jnp.int32, (SEG, CAP), 0)
    kio = lax.broadcasted_iota(jnp.int32, (SEG, CAP), 1)
    cols = lax.broadcasted_iota(jnp.int32, (CAP, SEG), 1)
    krow = lax.broadcasted_iota(jnp.int32, (CAP, SEG), 0)
    xgs = []
    ps = []
    for s in range(SPB):
        n = cnt_smem[SPB * i + s, 0]
        idxr = idx_ref[s]
        idxc = idxr.reshape(CAP, 1)
        gmat = ((cols == idxc) & (krow < n)).astype(jnp.bfloat16)
        xb = x_ref[s * SEG:(s + 1) * SEG, :].astype(jnp.bfloat16)
        xgs.append(_bf16_dot(gmat, xb))
        ps.append(((rows == idxr) & (kio < n)).astype(jnp.bfloat16))
    xg = jnp.concatenate(xgs, axis=0)
    soft = _bf16_dot(xg.astype(jnp.bfloat16), wt_ref[...]) + b_ref[...]
    normed = _ln(xg + soft, g_ref[...], be_ref[...])
    d = (normed - xg).astype(jnp.bfloat16)
    for s in range(SPB):
        o_ref[s * SEG:(s + 1) * SEG, :] = (
            x_ref[s * SEG:(s + 1) * SEG, :]
            + _bf16_dot(ps[s], d[s * CAP:(s + 1) * CAP])
        )


def _dense_body(x_ref, m_ref, wt_ref, b_ref, g_ref, be_ref, o_ref):
    x = x_ref[...]
    soft = _bf16_dot(x.astype(jnp.bfloat16), wt_ref[...]) + b_ref[...]
    normed = _ln(x + soft, g_ref[...], be_ref[...])
    o_ref[...] = jnp.where(m_ref[...] == 1, normed, x)


def kernel(batch_embeddings, position_mask, W, b, gamma, beta):
    B, S, Hh = batch_embeddings.shape
    N = B * S
    x = batch_embeddings.reshape(N, Hh)
    mflat = position_mask.reshape(N).astype(jnp.int32)
    wt = W.T.astype(jnp.bfloat16)
    b2 = b.reshape(1, Hh)
    g2 = gamma.reshape(1, Hh)
    be2 = beta.reshape(1, Hh)

    sc = pl.kernel(
        _sc_compact,
        out_type=(
            jax.ShapeDtypeStruct((NW, 1, CAP), jnp.int32),
            jax.ShapeDtypeStruct((NW, L), jnp.int32),
        ),
        mesh=_SC_MESH,
        compiler_params=pltpu.CompilerParams(needs_layout_passes=False),
        scratch_types=[
            pltpu.VMEM((SEG,), jnp.int32),
            pltpu.VMEM((CAP,), jnp.int32),
            pltpu.VMEM((L,), jnp.int32),
        ],
    )
    idx, cnt = sc(mflat)

    def sparse_path(ops):
        xx, cc, ii = ops
        return pl.pallas_call(
            _sparse_body,
            grid=(N // BLK,),
            in_specs=[
                pl.BlockSpec(memory_space=pltpu.SMEM),
                pl.BlockSpec((BLK, Hh), lambda i: (i, 0)),
                pl.BlockSpec((SPB, 1, CAP), lambda i: (i, 0, 0)),
                pl.BlockSpec((Hh, Hh), lambda i: (0, 0)),
                pl.BlockSpec((1, Hh), lambda i: (0, 0)),
                pl.BlockSpec((1, Hh), lambda i: (0, 0)),
                pl.BlockSpec((1, Hh), lambda i: (0, 0)),
            ],
            out_specs=pl.BlockSpec((BLK, Hh), lambda i: (i, 0)),
            out_shape=jax.ShapeDtypeStruct((N, Hh), jnp.float32),
            compiler_params=pltpu.CompilerParams(
                dimension_semantics=("parallel",),
            ),
        )(cc, xx, ii, wt, b2, g2, be2)

    def dense_path(ops):
        xx, cc, ii = ops
        return pl.pallas_call(
            _dense_body,
            grid=(N // BLK,),
            in_specs=[
                pl.BlockSpec((BLK, Hh), lambda i: (i, 0)),
                pl.BlockSpec((BLK, 1), lambda i: (i, 0)),
                pl.BlockSpec((Hh, Hh), lambda i: (0, 0)),
                pl.BlockSpec((1, Hh), lambda i: (0, 0)),
                pl.BlockSpec((1, Hh), lambda i: (0, 0)),
                pl.BlockSpec((1, Hh), lambda i: (0, 0)),
            ],
            out_specs=pl.BlockSpec((BLK, Hh), lambda i: (i, 0)),
            out_shape=jax.ShapeDtypeStruct((N, Hh), jnp.float32),
            compiler_params=pltpu.CompilerParams(
                dimension_semantics=("arbitrary",),
            ),
        )(xx, mflat.reshape(N, 1), wt, b2, g2, be2)

    overflow = jnp.any(cnt[:, 0] > CAP)
    out = lax.cond(overflow, dense_path, sparse_path, (x, cnt, idx))
    return out.reshape(B, S, Hh)
